# Initial kernel scaffold; baseline (speedup 1.0000x reference)
import jax
import jax.numpy as jnp
from jax.experimental import pallas as pl


def kernel(x, pos_table, gamma, beta, batch_size):
    raise NotImplementedError("write your pallas kernel here")


# speedup vs baseline: 3.4610x; 3.4610x over previous
import functools

import jax
import jax.numpy as jnp
from jax.experimental import pallas as pl

_EPS = 1e-5
_BLK_S = 512


def _ln_body(x_ref, pos_ref, gamma_ref, beta_ref, out_ref):
    h = x_ref[0] + pos_ref[...]
    mean = jnp.mean(h, axis=-1, keepdims=True)
    c = h - mean
    var = jnp.mean(c * c, axis=-1, keepdims=True)
    inv = jax.lax.rsqrt(var + _EPS)
    out_ref[0] = c * inv * gamma_ref[...] + beta_ref[...]


@jax.jit
def _ln_tc(x, pos_table, gamma, beta):
    b, s, d = x.shape
    grid = (s // _BLK_S, b)
    return pl.pallas_call(
        _ln_body,
        grid=grid,
        in_specs=[
            pl.BlockSpec((1, _BLK_S, d), lambda i, j: (j, i, 0)),
            pl.BlockSpec((_BLK_S, d), lambda i, j: (i, 0)),
            pl.BlockSpec((1, d), lambda i, j: (0, 0)),
            pl.BlockSpec((1, d), lambda i, j: (0, 0)),
        ],
        out_specs=pl.BlockSpec((1, _BLK_S, d), lambda i, j: (j, i, 0)),
        out_shape=jax.ShapeDtypeStruct((b, s, d), x.dtype),
    )(x, pos_table, gamma.reshape(1, d), beta.reshape(1, d))


def kernel(x, pos_table, gamma, beta, batch_size):
    return _ln_tc(x, pos_table, gamma, beta)


# speedup vs baseline: 4.0604x; 1.1732x over previous
import functools

import jax
import jax.numpy as jnp
from jax.experimental import pallas as pl

_EPS = 1e-5
_BLK_S = 1024


def _ln_body(x_ref, pos_ref, gamma_ref, beta_ref, out_ref):
    h = x_ref[0] + pos_ref[...]
    mean = jnp.mean(h, axis=-1, keepdims=True)
    c = h - mean
    var = jnp.mean(c * c, axis=-1, keepdims=True)
    inv = jax.lax.rsqrt(var + _EPS)
    out_ref[0] = c * inv * gamma_ref[...] + beta_ref[...]


@jax.jit
def _ln_tc(x, pos_table, gamma, beta):
    b, s, d = x.shape
    grid = (s // _BLK_S, b)
    return pl.pallas_call(
        _ln_body,
        grid=grid,
        in_specs=[
            pl.BlockSpec((1, _BLK_S, d), lambda i, j: (j, i, 0)),
            pl.BlockSpec((_BLK_S, d), lambda i, j: (i, 0)),
            pl.BlockSpec((1, d), lambda i, j: (0, 0)),
            pl.BlockSpec((1, d), lambda i, j: (0, 0)),
        ],
        out_specs=pl.BlockSpec((1, _BLK_S, d), lambda i, j: (j, i, 0)),
        out_shape=jax.ShapeDtypeStruct((b, s, d), x.dtype),
    )(x, pos_table, gamma.reshape(1, d), beta.reshape(1, d))


def kernel(x, pos_table, gamma, beta, batch_size):
    return _ln_tc(x, pos_table, gamma, beta)


# speedup vs baseline: 4.4154x; 1.0874x over previous
import functools

import jax
import jax.numpy as jnp
from jax.experimental import pallas as pl

_EPS = 1e-5
_BLK_S = 2048


def _ln_body(x_ref, pos_ref, gamma_ref, beta_ref, out_ref):
    h = x_ref[0] + pos_ref[...]
    mean = jnp.mean(h, axis=-1, keepdims=True)
    c = h - mean
    var = jnp.mean(c * c, axis=-1, keepdims=True)
    inv = jax.lax.rsqrt(var + _EPS)
    out_ref[0] = c * inv * gamma_ref[...] + beta_ref[...]


@jax.jit
def _ln_tc(x, pos_table, gamma, beta):
    b, s, d = x.shape
    grid = (s // _BLK_S, b)
    return pl.pallas_call(
        _ln_body,
        grid=grid,
        in_specs=[
            pl.BlockSpec((1, _BLK_S, d), lambda i, j: (j, i, 0)),
            pl.BlockSpec((_BLK_S, d), lambda i, j: (i, 0)),
            pl.BlockSpec((1, d), lambda i, j: (0, 0)),
            pl.BlockSpec((1, d), lambda i, j: (0, 0)),
        ],
        out_specs=pl.BlockSpec((1, _BLK_S, d), lambda i, j: (j, i, 0)),
        out_shape=jax.ShapeDtypeStruct((b, s, d), x.dtype),
    )(x, pos_table, gamma.reshape(1, d), beta.reshape(1, d))


def kernel(x, pos_table, gamma, beta, batch_size):
    return _ln_tc(x, pos_table, gamma, beta)
